# baseline (speedup 1.0000x reference)
import functools

import jax
import jax.numpy as jnp
from jax.experimental import pallas as pl
from jax.experimental.pallas import tpu as pltpu

_NX, _NY, _NZ = 128, 128, 16
_E = 128
_C = 256
_YZ = _NY * _NZ
_N_CAM = 6
_HW = 32 * 88
_XB = 8


def _prep_kernel(feats_ref, wqT_ref, wv_ref, bT_ref, coefT_ref, acc_ref):
    i = pl.program_id(0)

    @pl.when(i == 0)
    def _():
        acc_ref[...] = jnp.zeros_like(acc_ref)

    acc_ref[...] += jnp.sum(feats_ref[0], axis=-1, keepdims=True)

    @pl.when(i == _N_CAM - 1)
    def _():
        dT = jax.lax.dot_general(
            wv_ref[...], acc_ref[...],
            (((0,), (0,)), ((), ())),
            preferred_element_type=jnp.float32,
        ) * (1.0 / (_N_CAM * _HW))
        wqT = wqT_ref[...]
        axc = 0.4 * wqT[:, 0:1]
        ayc = 0.4 * wqT[:, 1:2]
        azc = 0.4 * wqT[:, 2:3]
        dcol = (dT + bT_ref[...]
                - 25.6 * (wqT[:, 0:1] + wqT[:, 1:2]) - 3.2 * wqT[:, 2:3])
        coefT_ref[:, 0:1] = axc
        coefT_ref[:, 1:2] = ayc
        coefT_ref[:, 2:3] = azc
        coefT_ref[:, 3:4] = dcol
        coefT_ref[:, 4:8] = jnp.zeros((_E, 4), jnp.float32)


def _fill_kernel(coefT_ref, prop_ref, out_ref):
    i = pl.program_id(0)
    coefT = coefT_ref[...]
    ax = coefT[:, 0:1]
    ay = coefT[:, 1:2]
    az = coefT[:, 2:3]
    d = coefT[:, 3:4]
    yz = jax.lax.broadcasted_iota(jnp.int32, (_E, _YZ), 1)
    t = (ay * (yz // _NZ).astype(jnp.float32)
         + az * (yz % _NZ).astype(jnp.float32) + d)
    for x in range(_XB):
        xs = (i * _XB + x).astype(jnp.float32)
        mf = (prop_ref[x:x + 1, :] > 0).astype(jnp.float32)
        out_ref[:, x, :] = mf * (t + ax * xs)


@functools.partial(jax.jit, static_argnames=())
def kernel(mlvl_feats_0, proposal, W_q, W_v, b):
    feats = mlvl_feats_0.reshape(_N_CAM, _C, _HW)
    wqT = W_q.T
    bT = b.reshape(_E, 1)
    coefT = pl.pallas_call(
        _prep_kernel,
        grid=(_N_CAM,),
        in_specs=[
            pl.BlockSpec((1, _C, _HW), lambda i: (i, 0, 0)),
            pl.BlockSpec((_E, 3), lambda i: (0, 0)),
            pl.BlockSpec((_C, _E), lambda i: (0, 0)),
            pl.BlockSpec((_E, 1), lambda i: (0, 0)),
        ],
        out_specs=pl.BlockSpec((_E, 8), lambda i: (0, 0)),
        out_shape=jax.ShapeDtypeStruct((_E, 8), jnp.float32),
        scratch_shapes=[pltpu.VMEM((_C, 1), jnp.float32)],
    )(feats, wqT, W_v, bT)

    prop2d = proposal.reshape(_NX, _YZ)
    vol = pl.pallas_call(
        _fill_kernel,
        grid=(_NX // _XB,),
        in_specs=[
            pl.BlockSpec((_E, 8), lambda i: (0, 0)),
            pl.BlockSpec((_XB, _YZ), lambda i: (i, 0)),
        ],
        out_specs=pl.BlockSpec((_E, _XB, _YZ), lambda i: (0, i, 0)),
        out_shape=jax.ShapeDtypeStruct((_E, _NX, _YZ), jnp.float32),
    )(coefT, prop2d)
    return vol.reshape(1, _E, _NX, _NY, _NZ)


# speedup vs baseline: 1.0617x; 1.0617x over previous
import functools

import jax
import jax.numpy as jnp
from jax.experimental import pallas as pl
from jax.experimental.pallas import tpu as pltpu

_NX, _NY, _NZ = 128, 128, 16
_E = 128
_C = 256
_YZ = _NY * _NZ
_N_CAM = 6
_HW = 32 * 88
_XB = 8


def _prep_kernel(feats_ref, wqT_ref, wv_ref, bT_ref, coefT_ref, acc_ref):
    i = pl.program_id(0)

    @pl.when(i == 0)
    def _():
        acc_ref[...] = jnp.zeros_like(acc_ref)

    acc_ref[...] += jnp.sum(feats_ref[0], axis=-1, keepdims=True)

    @pl.when(i == _N_CAM - 1)
    def _():
        dT = jax.lax.dot_general(
            wv_ref[...], acc_ref[...],
            (((0,), (0,)), ((), ())),
            preferred_element_type=jnp.float32,
        ) * (1.0 / (_N_CAM * _HW))
        wqT = wqT_ref[...]
        axc = 0.4 * wqT[:, 0:1]
        ayc = 0.4 * wqT[:, 1:2]
        azc = 0.4 * wqT[:, 2:3]
        dcol = (dT + bT_ref[...]
                - 25.6 * (wqT[:, 0:1] + wqT[:, 1:2]) - 3.2 * wqT[:, 2:3])
        coefT_ref[:, 0:1] = axc
        coefT_ref[:, 1:2] = ayc
        coefT_ref[:, 2:3] = azc
        coefT_ref[:, 3:4] = dcol
        coefT_ref[:, 4:8] = jnp.zeros((_E, 4), jnp.float32)


def _fill_kernel(coefT_ref, prop_ref, out_ref):
    i = pl.program_id(0)
    coefT = coefT_ref[...]
    ax = coefT[:, 0:1]
    ay = coefT[:, 1:2]
    az = coefT[:, 2:3]
    d = coefT[:, 3:4]
    yz = jax.lax.broadcasted_iota(jnp.int32, (_E, _YZ), 1)
    t = (ay * (yz // _NZ).astype(jnp.float32)
         + az * (yz % _NZ).astype(jnp.float32) + d)
    xg = (jax.lax.broadcasted_iota(jnp.int32, (_E, _XB), 1)
          + i * _XB).astype(jnp.float32)
    u = ax * xg
    keep = prop_ref[...] > 0
    val = u[:, :, None] + t[:, None, :]
    out_ref[...] = jnp.where(keep[None], val, 0.0)


@functools.partial(jax.jit, static_argnames=())
def kernel(mlvl_feats_0, proposal, W_q, W_v, b):
    feats = mlvl_feats_0.reshape(_N_CAM, _C, _HW)
    wqT = W_q.T
    bT = b.reshape(_E, 1)
    coefT = pl.pallas_call(
        _prep_kernel,
        grid=(_N_CAM,),
        in_specs=[
            pl.BlockSpec((1, _C, _HW), lambda i: (i, 0, 0)),
            pl.BlockSpec((_E, 3), lambda i: (0, 0)),
            pl.BlockSpec((_C, _E), lambda i: (0, 0)),
            pl.BlockSpec((_E, 1), lambda i: (0, 0)),
        ],
        out_specs=pl.BlockSpec((_E, 8), lambda i: (0, 0)),
        out_shape=jax.ShapeDtypeStruct((_E, 8), jnp.float32),
        scratch_shapes=[pltpu.VMEM((_C, 1), jnp.float32)],
    )(feats, wqT, W_v, bT)

    prop2d = proposal.reshape(_NX, _YZ)
    vol = pl.pallas_call(
        _fill_kernel,
        grid=(_NX // _XB,),
        in_specs=[
            pl.BlockSpec((_E, 8), lambda i: (0, 0)),
            pl.BlockSpec((_XB, _YZ), lambda i: (i, 0)),
        ],
        out_specs=pl.BlockSpec((_E, _XB, _YZ), lambda i: (0, i, 0)),
        out_shape=jax.ShapeDtypeStruct((_E, _NX, _YZ), jnp.float32),
    )(coefT, prop2d)
    return vol.reshape(1, _E, _NX, _NY, _NZ)


# speedup vs baseline: 1.4941x; 1.4072x over previous
import functools

import jax
import jax.numpy as jnp
from jax.experimental import pallas as pl
from jax.experimental.pallas import tpu as pltpu

_NX, _NY, _NZ = 128, 128, 16
_E = 128
_C = 256
_N_VOX = _NX * _NY * _NZ
_N_CAM = 6
_H, _W = 32, 88
_VB = 4096
_NBLK = _N_VOX // _VB


def _prep_kernel(feats_ref, wq_ref, wv_ref, b_ref, a_ref, acc_ref):
    i = pl.program_id(0)

    @pl.when(i == 0)
    def _():
        acc_ref[...] = jnp.zeros_like(acc_ref)

    s = jnp.sum(feats_ref[0, 0], axis=(1, 2))
    acc_ref[...] += s.reshape(1, _C)

    @pl.when(i == _N_CAM - 1)
    def _():
        ctx = acc_ref[...] * (1.0 / (_N_CAM * _H * _W))
        d = jax.lax.dot_general(
            ctx, wv_ref[...], (((1,), (0,)), ((), ())),
            preferred_element_type=jnp.float32,
        )
        wq = wq_ref[...]
        a_ref[0:1, :] = 0.4 * wq[0:1, :]
        a_ref[1:2, :] = 0.4 * wq[1:2, :]
        a_ref[2:3, :] = 0.4 * wq[2:3, :]
        a_ref[3:4, :] = (d + b_ref[...]
                         - 25.6 * (wq[0:1, :] + wq[1:2, :])
                         - 3.2 * wq[2:3, :])
        a_ref[4:8, :] = jnp.zeros((4, _E), jnp.float32)


def _fill_kernel(a_ref, prop_ref, out_ref):
    i = pl.program_id(0)
    v = jax.lax.broadcasted_iota(jnp.int32, (1, _VB), 1) + i * _VB
    m = (prop_ref[0] > 0).astype(jnp.float32)
    xm = (v >> 11).astype(jnp.float32) * m
    ym = ((v >> 4) & 127).astype(jnp.float32) * m
    zm = (v & 15).astype(jnp.float32) * m
    p = jnp.concatenate(
        [xm, ym, zm, m, jnp.zeros((4, _VB), jnp.float32)], axis=0)
    out_ref[...] = jax.lax.dot_general(
        p, a_ref[...], (((0,), (0,)), ((), ())),
        preferred_element_type=jnp.float32,
    )


@functools.partial(jax.jit, static_argnames=())
def kernel(mlvl_feats_0, proposal, W_q, W_v, b):
    coefA = pl.pallas_call(
        _prep_kernel,
        grid=(_N_CAM,),
        in_specs=[
            pl.BlockSpec((1, 1, _C, _H, _W), lambda i: (0, i, 0, 0, 0)),
            pl.BlockSpec((3, _E), lambda i: (0, 0)),
            pl.BlockSpec((_C, _E), lambda i: (0, 0)),
            pl.BlockSpec((1, _E), lambda i: (0, 0)),
        ],
        out_specs=pl.BlockSpec((8, _E), lambda i: (0, 0)),
        out_shape=jax.ShapeDtypeStruct((8, _E), jnp.float32),
        scratch_shapes=[pltpu.VMEM((1, _C), jnp.float32)],
    )(mlvl_feats_0, W_q, W_v, b.reshape(1, _E))

    prop3d = proposal.reshape(_NBLK, 1, _VB)
    vol = pl.pallas_call(
        _fill_kernel,
        grid=(_NBLK,),
        in_specs=[
            pl.BlockSpec((8, _E), lambda i: (0, 0)),
            pl.BlockSpec((1, 1, _VB), lambda i: (i, 0, 0)),
        ],
        out_specs=pl.BlockSpec((_VB, _E), lambda i: (i, 0)),
        out_shape=jax.ShapeDtypeStruct((_N_VOX, _E), jnp.float32),
    )(coefA, prop3d)
    vol = vol.reshape(_NX, _NY, _NZ, _E)
    return jnp.transpose(vol, (3, 0, 1, 2))[None]


# speedup vs baseline: 1.8428x; 1.2334x over previous
import functools

import jax
import jax.numpy as jnp
from jax.experimental import pallas as pl
from jax.experimental.pallas import tpu as pltpu

_NX, _NY, _NZ = 128, 128, 16
_E = 128
_C = 256
_N_VOX = _NX * _NY * _NZ
_N_CAM = 6
_H, _W = 32, 88
_VB = 8192
_NBLK = _N_VOX // _VB


def _prep_kernel(feats_ref, wq_ref, wv_ref, b_ref, a_ref, acc_ref):
    i = pl.program_id(0)

    @pl.when(i == 0)
    def _():
        acc_ref[...] = jnp.zeros_like(acc_ref)

    s = jnp.sum(feats_ref[0, 0], axis=(0, 1))
    acc_ref[...] += s.reshape(1, _C)

    @pl.when(i == _N_CAM - 1)
    def _():
        ctx = acc_ref[...] * (1.0 / (_N_CAM * _H * _W))
        d = jax.lax.dot_general(
            ctx, wv_ref[...], (((1,), (0,)), ((), ())),
            preferred_element_type=jnp.float32,
        )
        wq = wq_ref[...]
        a_ref[0:1, :] = 0.4 * wq[0:1, :]
        a_ref[1:2, :] = 0.4 * wq[1:2, :]
        a_ref[2:3, :] = 0.4 * wq[2:3, :]
        a_ref[3:4, :] = (d + b_ref[...]
                         - 25.6 * (wq[0:1, :] + wq[1:2, :])
                         - 3.2 * wq[2:3, :])
        a_ref[4:8, :] = jnp.zeros((4, _E), jnp.float32)


def _fill_kernel(a_ref, prop_ref, out_ref):
    i = pl.program_id(0)
    v = jax.lax.broadcasted_iota(jnp.int32, (1, _VB), 1) + i * _VB
    m = (prop_ref[0] > 0).astype(jnp.float32)
    xm = (v >> 11).astype(jnp.float32) * m
    ym = ((v >> 4) & 127).astype(jnp.float32) * m
    zm = (v & 15).astype(jnp.float32) * m
    p = jnp.concatenate(
        [xm, ym, zm, m, jnp.zeros((4, _VB), jnp.float32)], axis=0)
    out_ref[...] = jax.lax.dot_general(
        p, a_ref[...], (((0,), (0,)), ((), ())),
        preferred_element_type=jnp.float32,
    )


@functools.partial(jax.jit, static_argnames=())
def kernel(mlvl_feats_0, proposal, W_q, W_v, b):
    feats_t = jnp.transpose(mlvl_feats_0, (0, 1, 3, 4, 2))
    coefA = pl.pallas_call(
        _prep_kernel,
        grid=(_N_CAM,),
        in_specs=[
            pl.BlockSpec((1, 1, _H, _W, _C), lambda i: (0, i, 0, 0, 0)),
            pl.BlockSpec((3, _E), lambda i: (0, 0)),
            pl.BlockSpec((_C, _E), lambda i: (0, 0)),
            pl.BlockSpec((1, _E), lambda i: (0, 0)),
        ],
        out_specs=pl.BlockSpec((8, _E), lambda i: (0, 0)),
        out_shape=jax.ShapeDtypeStruct((8, _E), jnp.float32),
        scratch_shapes=[pltpu.VMEM((1, _C), jnp.float32)],
    )(feats_t, W_q, W_v, b.reshape(1, _E))

    prop3d = proposal.reshape(_NBLK, 1, _VB)
    vol = pl.pallas_call(
        _fill_kernel,
        grid=(_NBLK,),
        in_specs=[
            pl.BlockSpec((8, _E), lambda i: (0, 0)),
            pl.BlockSpec((1, 1, _VB), lambda i: (i, 0, 0)),
        ],
        out_specs=pl.BlockSpec((_VB, _E), lambda i: (i, 0)),
        out_shape=jax.ShapeDtypeStruct((_N_VOX, _E), jnp.float32),
    )(coefA, prop3d)
    vol = vol.reshape(_NX, _NY, _NZ, _E)
    return jnp.transpose(vol, (3, 0, 1, 2))[None]


# speedup vs baseline: 2.7663x; 1.5012x over previous
import functools

import jax
import jax.numpy as jnp
from jax.experimental import pallas as pl
from jax.experimental.pallas import tpu as pltpu

_NX, _NY, _NZ = 128, 128, 16
_E = 128
_C = 256
_N_VOX = _NX * _NY * _NZ
_N_CAM = 6
_H, _W = 32, 88
_XZ = _NX * _NZ


def _prep_kernel(feats_ref, wq_ref, wv_ref, b_ref, a_ref, acc_ref):
    i = pl.program_id(0)

    @pl.when(i == 0)
    def _():
        acc_ref[...] = jnp.zeros_like(acc_ref)

    s = jnp.sum(feats_ref[0, 0], axis=(0, 1))
    acc_ref[...] += s.reshape(1, _C)

    @pl.when(i == _N_CAM - 1)
    def _():
        ctx = acc_ref[...] * (1.0 / (_N_CAM * _H * _W))
        d = jax.lax.dot_general(
            ctx, wv_ref[...], (((1,), (0,)), ((), ())),
            preferred_element_type=jnp.float32,
        )
        wq = wq_ref[...]
        a_ref[0:1, :] = 0.4 * wq[0:1, :]
        a_ref[1:2, :] = 0.4 * wq[1:2, :]
        a_ref[2:3, :] = 0.4 * wq[2:3, :]
        a_ref[3:4, :] = (d + b_ref[...]
                         - 25.6 * (wq[0:1, :] + wq[1:2, :])
                         - 3.2 * wq[2:3, :])
        a_ref[4:8, :] = jnp.zeros((4, _E), jnp.float32)


def _fill_kernel(a_ref, mf_ref, out_ref):
    e = pl.program_id(0)
    ax = a_ref[0, e]
    ay = a_ref[1, e]
    az = a_ref[2, e]
    d = a_ref[3, e]
    c = jax.lax.broadcasted_iota(jnp.int32, (1, _XZ), 1)
    xr = (c >> 4).astype(jnp.float32)
    zr = (c & 15).astype(jnp.float32)
    p2 = jnp.concatenate(
        [xr, zr, jnp.ones((1, _XZ), jnp.float32),
         jnp.zeros((5, _XZ), jnp.float32)], axis=0)
    yg = jax.lax.broadcasted_iota(jnp.int32, (1, _NY), 1).astype(jnp.float32)
    a2 = jnp.concatenate(
        [jnp.full((1, _NY), ax), jnp.full((1, _NY), az), d + ay * yg,
         jnp.zeros((5, _NY), jnp.float32)], axis=0)
    o = jax.lax.dot_general(
        p2, a2, (((0,), (0,)), ((), ())),
        preferred_element_type=jnp.float32,
    )
    out_ref[...] = o * mf_ref[...]


@functools.partial(jax.jit, static_argnames=())
def kernel(mlvl_feats_0, proposal, W_q, W_v, b):
    feats_t = jnp.transpose(mlvl_feats_0, (0, 1, 3, 4, 2))
    coefA = pl.pallas_call(
        _prep_kernel,
        grid=(_N_CAM,),
        in_specs=[
            pl.BlockSpec((1, 1, _H, _W, _C), lambda i: (0, i, 0, 0, 0)),
            pl.BlockSpec((3, _E), lambda i: (0, 0)),
            pl.BlockSpec((_C, _E), lambda i: (0, 0)),
            pl.BlockSpec((1, _E), lambda i: (0, 0)),
        ],
        out_specs=pl.BlockSpec((8, _E), lambda i: (0, 0)),
        out_shape=jax.ShapeDtypeStruct((8, _E), jnp.float32),
        scratch_shapes=[pltpu.VMEM((1, _C), jnp.float32)],
    )(feats_t, W_q, W_v, b.reshape(1, _E))

    mf = ((proposal > 0).astype(jnp.float32)
          .reshape(_NX, _NY, _NZ).transpose(0, 2, 1).reshape(_XZ, _NY))
    vol = pl.pallas_call(
        _fill_kernel,
        grid=(_E,),
        in_specs=[
            pl.BlockSpec(memory_space=pltpu.SMEM),
            pl.BlockSpec((_XZ, _NY), lambda e: (0, 0)),
        ],
        out_specs=pl.BlockSpec((_XZ, _NY), lambda e: (e, 0)),
        out_shape=jax.ShapeDtypeStruct((_E * _XZ, _NY), jnp.float32),
    )(coefA, mf)
    v4 = vol.reshape(_E, _NX, _NZ, _NY)
    return jnp.transpose(v4, (0, 1, 3, 2))[None]


# speedup vs baseline: 5.0637x; 1.8305x over previous
import functools

import jax
import jax.numpy as jnp
from jax.experimental import pallas as pl
from jax.experimental.pallas import tpu as pltpu

_NX, _NY, _NZ = 128, 128, 16
_E = 128
_C = 256
_N_VOX = _NX * _NY * _NZ
_N_CAM = 6
_H, _W = 32, 88
_XZ = _NX * _NZ


def _prep_kernel(feats_ref, wq_ref, wv_ref, b_ref, a_ref, acc_ref):
    i = pl.program_id(0)

    @pl.when(i == 0)
    def _():
        acc_ref[...] = jnp.zeros_like(acc_ref)

    s = jnp.sum(feats_ref[0, 0], axis=(0, 1))
    acc_ref[...] += s.reshape(1, _C)

    @pl.when(i == _N_CAM - 1)
    def _():
        ctx = acc_ref[...] * (1.0 / (_N_CAM * _H * _W))
        d = jax.lax.dot_general(
            ctx, wv_ref[...], (((1,), (0,)), ((), ())),
            preferred_element_type=jnp.float32,
        )
        wq = wq_ref[...]
        a_ref[0:1, :] = 0.4 * wq[0:1, :]
        a_ref[1:2, :] = 0.4 * wq[1:2, :]
        a_ref[2:3, :] = 0.4 * wq[2:3, :]
        a_ref[3:4, :] = (d + b_ref[...]
                         - 25.6 * (wq[0:1, :] + wq[1:2, :])
                         - 3.2 * wq[2:3, :])
        a_ref[4:8, :] = jnp.zeros((4, _E), jnp.float32)


_EB = 8


def _fill_kernel(a_ref, mf_ref, out_ref):
    i = pl.program_id(0)
    c = jax.lax.broadcasted_iota(jnp.int32, (1, _XZ), 1)
    xr = (c >> 4).astype(jnp.float32)
    zr = (c & 15).astype(jnp.float32)
    p2 = jnp.concatenate(
        [xr, zr, jnp.ones((1, _XZ), jnp.float32),
         jnp.zeros((5, _XZ), jnp.float32)], axis=0)
    yg = jax.lax.broadcasted_iota(jnp.int32, (1, _NY), 1).astype(jnp.float32)
    for j in range(_EB):
        e = i * _EB + j
        ax = a_ref[0, e]
        ay = a_ref[1, e]
        az = a_ref[2, e]
        d = a_ref[3, e]
        a2 = jnp.concatenate(
            [jnp.full((1, _NY), ax), jnp.full((1, _NY), az), d + ay * yg,
             jnp.zeros((5, _NY), jnp.float32)], axis=0)
        o = jax.lax.dot_general(
            p2, a2, (((0,), (0,)), ((), ())),
            preferred_element_type=jnp.float32,
        )
        out_ref[j * _XZ:(j + 1) * _XZ, :] = o * mf_ref[...]


@functools.partial(jax.jit, static_argnames=())
def kernel(mlvl_feats_0, proposal, W_q, W_v, b):
    feats_t = jnp.transpose(mlvl_feats_0, (0, 1, 3, 4, 2))
    coefA = pl.pallas_call(
        _prep_kernel,
        grid=(_N_CAM,),
        in_specs=[
            pl.BlockSpec((1, 1, _H, _W, _C), lambda i: (0, i, 0, 0, 0)),
            pl.BlockSpec((3, _E), lambda i: (0, 0)),
            pl.BlockSpec((_C, _E), lambda i: (0, 0)),
            pl.BlockSpec((1, _E), lambda i: (0, 0)),
        ],
        out_specs=pl.BlockSpec((8, _E), lambda i: (0, 0)),
        out_shape=jax.ShapeDtypeStruct((8, _E), jnp.float32),
        scratch_shapes=[pltpu.VMEM((1, _C), jnp.float32)],
    )(feats_t, W_q, W_v, b.reshape(1, _E))

    mf = ((proposal > 0).astype(jnp.float32)
          .reshape(_NX, _NY, _NZ).transpose(0, 2, 1).reshape(_XZ, _NY))
    vol = pl.pallas_call(
        _fill_kernel,
        grid=(_E // _EB,),
        in_specs=[
            pl.BlockSpec(memory_space=pltpu.SMEM),
            pl.BlockSpec((_XZ, _NY), lambda i: (0, 0)),
        ],
        out_specs=pl.BlockSpec((_EB * _XZ, _NY), lambda i: (i, 0)),
        out_shape=jax.ShapeDtypeStruct((_E * _XZ, _NY), jnp.float32),
    )(coefA, mf)
    v4 = vol.reshape(_E, _NX, _NZ, _NY)
    return jnp.transpose(v4, (0, 1, 3, 2))[None]
